# baseline (speedup 1.0000x reference)
import functools

import jax
import jax.numpy as jnp
from jax.experimental import pallas as pl
from jax.experimental.pallas import tpu as pltpu


def _make_expand_body(word_len, c_block):
    def _expand_body(pos_ref, w_ref, out_ref):
        d = w_ref.shape[1]
        max_depth = word_len // 2

        c = jax.lax.broadcasted_iota(jnp.int32, (1, c_block), 1)
        c = c + pl.program_id(1) * c_block
        k = c // d
        t = k // 2
        par = k % 2
        e = k % max_depth

        w = jnp.tanh(w_ref[...])
        scale = jnp.sqrt((1.0 - w * w) * (d / 2.0))
        n_rep = c_block // d
        wt = jnp.concatenate([w] * n_rep, axis=1)
        st = jnp.concatenate([scale] * n_rep, axis=1)
        w2 = wt * wt
        w4 = w2 * w2
        w8 = w4 * w4
        e2 = e[0]
        v = st
        v = v * jnp.where((e2 & 1) != 0, wt, 1.0)
        v = v * jnp.where((e2 & 2) != 0, w2, 1.0)
        v = v * jnp.where((e2 & 4) != 0, w4, 1.0)
        v = v * jnp.where((e2 & 8) != 0, w8, 1.0)
        v = v[None]

        m = jnp.maximum(pos_ref[...], 1)
        sh = jnp.right_shift(m[:, :, None], t[None])
        bit = (sh >= 2) & ((sh & 1) == par[None])
        out_ref[...] = jnp.where(bit, v, 0.0)

    return _expand_body


@functools.partial(jax.jit, static_argnames=("block_b", "n_c_blocks", "word_len"))
def _expand(pos_t, weight_row, block_b=128, n_c_blocks=5, word_len=20):
    s, b = pos_t.shape
    d = weight_row.shape[1]
    c = word_len * d
    c_block = c // n_c_blocks
    grid = (b // block_b, n_c_blocks)
    return pl.pallas_call(
        _make_expand_body(word_len, c_block),
        grid=grid,
        in_specs=[
            pl.BlockSpec((s, block_b), lambda i, j: (0, i)),
            pl.BlockSpec((1, d), lambda i, j: (0, 0)),
        ],
        out_specs=pl.BlockSpec((s, block_b, c_block), lambda i, j: (0, i, j)),
        out_shape=jax.ShapeDtypeStruct((s, b, c), jnp.float32),
        compiler_params=pltpu.CompilerParams(
            dimension_semantics=("arbitrary", "arbitrary"),
        ),
    )(pos_t, weight_row)


def kernel(positions, weight, paths):
    d = weight.shape[0]
    word_len = paths.shape[1]
    weight_row = weight.reshape(1, d)
    out_t = _expand(positions.T, weight_row, word_len=word_len)
    return jnp.transpose(out_t, (1, 0, 2))


# speedup vs baseline: 1.3182x; 1.3182x over previous
import functools

import jax
import jax.numpy as jnp
from jax.experimental import pallas as pl
from jax.experimental.pallas import tpu as pltpu


def _make_expand_body(word_len, c_block):
    def _expand_body(pos_ref, w_ref, out_ref):
        d = w_ref.shape[1]
        max_depth = word_len // 2

        c = jax.lax.broadcasted_iota(jnp.int32, (1, c_block), 1)
        c = c + pl.program_id(1) * c_block
        k = c // d
        t = k // 2
        par = k % 2
        e = k % max_depth

        w = jnp.tanh(w_ref[...])
        scale = jnp.sqrt((1.0 - w * w) * (d / 2.0))
        n_rep = c_block // d
        wt = jnp.concatenate([w] * n_rep, axis=1)
        st = jnp.concatenate([scale] * n_rep, axis=1)
        w2 = wt * wt
        w4 = w2 * w2
        w8 = w4 * w4
        e2 = e[0]
        v = st
        v = v * jnp.where((e2 & 1) != 0, wt, 1.0)
        v = v * jnp.where((e2 & 2) != 0, w2, 1.0)
        v = v * jnp.where((e2 & 4) != 0, w4, 1.0)
        v = v * jnp.where((e2 & 8) != 0, w8, 1.0)
        v = v[None]

        m = jnp.maximum(pos_ref[...], 1)
        sh = jnp.right_shift(m[:, :, None], t[None])
        bit = (sh >= 2) & ((sh & 1) == par[None])
        out_ref[...] = jnp.where(bit, v, 0.0)

    return _expand_body


@functools.partial(jax.jit, static_argnames=("block_b", "n_c_blocks", "word_len"))
def _expand(pos_t, weight_row, block_b=128, n_c_blocks=1, word_len=20):
    s, b = pos_t.shape
    d = weight_row.shape[1]
    c = word_len * d
    c_block = c // n_c_blocks
    grid = (b // block_b, n_c_blocks)
    return pl.pallas_call(
        _make_expand_body(word_len, c_block),
        grid=grid,
        in_specs=[
            pl.BlockSpec((s, block_b), lambda i, j: (0, i)),
            pl.BlockSpec((1, d), lambda i, j: (0, 0)),
        ],
        out_specs=pl.BlockSpec((s, block_b, c_block), lambda i, j: (0, i, j)),
        out_shape=jax.ShapeDtypeStruct((s, b, c), jnp.float32),
        compiler_params=pltpu.CompilerParams(
            dimension_semantics=("arbitrary", "arbitrary"),
        ),
    )(pos_t, weight_row)


def kernel(positions, weight, paths):
    d = weight.shape[0]
    word_len = paths.shape[1]
    weight_row = weight.reshape(1, d)
    out_t = _expand(positions.T, weight_row, word_len=word_len)
    return jnp.transpose(out_t, (1, 0, 2))
